# baseline (speedup 1.0000x reference)
import functools

import jax
import jax.numpy as jnp
from jax import lax
from jax.experimental import pallas as pl
from jax.experimental.pallas import tpu as pltpu
from jax.experimental.pallas import tpu_sc as plsc

N, D, M, C, E, K = 4096, 256, 8192, 6, 256, 5
BA = 128
BC = 256
BP = 512
NEG = -1e30
BIGI = 2**30
f32 = jnp.float32
bf16 = jnp.bfloat16


def _bmm(a, b, dims):
    return jax.lax.dot_general(
        a.astype(bf16), b.astype(bf16), dims,
        preferred_element_type=f32)


def _fw_body(f_ref, we_ref, fw_ref):
    fw_ref[...] = _bmm(f_ref[...], we_ref[...],
                       (((1,), (0,)), ((), ()))).astype(bf16)


def _c16_body(tk_ref, tl_ref, mfl_ref, wd_ref, c16_ref):
    tv = jax.lax.dot_general(
        tk_ref[...], wd_ref[...], (((1,), (0,)), ((), ())),
        precision=jax.lax.Precision.HIGHEST,
        preferred_element_type=f32) * 0.5
    lab = tl_ref[...]
    m = jnp.max(lab, axis=1, keepdims=True)
    ci = jax.lax.broadcasted_iota(jnp.int32, lab.shape, 1)
    ji = jnp.min(jnp.where(lab == m, ci, BIGI), axis=1, keepdims=True)
    oh = (ci == ji).astype(f32)
    lg = jax.lax.dot_general(
        oh, mfl_ref[...], (((1,), (0,)), ((), ())),
        precision=jax.lax.Precision.HIGHEST,
        preferred_element_type=f32) * 0.1
    z = jnp.zeros((lab.shape[0], 128 - 2 * C), f32)
    c16_ref[...] = jnp.concatenate([tv, lg, z], axis=1)


def _main_body(adj_ref, fw_ref, tk_ref, wd_ref,
               idx_ref, u0_ref, abf_ref):
    a = adj_ref[...]
    rs = jnp.sum(a, axis=1, keepdims=True) + 1e-8
    ab = (a / rs).astype(bf16)
    abf_ref[...] = ab
    p = jnp.maximum(
        jax.lax.dot_general(ab, fw_ref[...], (((1,), (0,)), ((), ())),
                            preferred_element_type=f32), 0.0)
    u0_ref[...] = jax.lax.dot_general(
        p, wd_ref[...], (((1,), (0,)), ((), ())),
        precision=jax.lax.Precision.HIGHEST,
        preferred_element_type=f32)
    s = jax.lax.dot_general(
        p.astype(bf16), tk_ref[...], (((1,), (1,)), ((), ())),
        preferred_element_type=f32)
    col = jax.lax.broadcasted_iota(jnp.int32, s.shape, 1)
    m = jnp.max(s, axis=1, keepdims=True)
    idxs = []
    for k in range(K):
        eq = s == m
        idxs.append(jnp.min(jnp.where(eq, col, BIGI), axis=1, keepdims=True))
        s = jnp.where(eq, NEG, s)
        if k < K - 1:
            m = jnp.max(s, axis=1, keepdims=True)
    idx_ref[...] = jnp.concatenate(idxs, axis=1)


_SC_INFO = plsc.get_sparse_core_info()
_NW = _SC_INFO.num_cores * _SC_INFO.num_subcores
_ROWS_W = N // _NW


def _sc_gather_body(idx_hbm, c16_hbm, out_hbm, idx_v, rows_v, acc_v, sem):
    wid = lax.axis_index("s") * _SC_INFO.num_cores + lax.axis_index("c")
    pltpu.sync_copy(idx_hbm.at[wid], idx_v)
    for k in range(K):
        pltpu.async_copy(c16_hbm.at[idx_v.at[k]], rows_v.at[k], sem).wait()
    for i in range(_ROWS_W):
        v = rows_v[0, i, pl.ds(0, 16)]
        for k in range(1, K):
            v = v + rows_v[k, i, pl.ds(0, 16)]
        acc_v[i, :] = v
    pltpu.sync_copy(acc_v, out_hbm.at[pl.ds(wid * _ROWS_W, _ROWS_W)])


@functools.partial(
    pl.kernel,
    mesh=plsc.VectorSubcoreMesh(core_axis_name="c", subcore_axis_name="s"),
    out_type=jax.ShapeDtypeStruct((N, 16), f32),
    scratch_types=[
        pltpu.VMEM((8, _ROWS_W), jnp.int32),
        pltpu.VMEM((K, _ROWS_W, 128), f32),
        pltpu.VMEM((_ROWS_W, 16), f32),
        pltpu.SemaphoreType.DMA,
    ],
)
def _sc_gather(idx_hbm, c16_hbm, out_hbm, idx_v, rows_v, acc_v, sem):
    _sc_gather_body(idx_hbm, c16_hbm, out_hbm, idx_v, rows_v, acc_v, sem)


def _chain_body(abf_ref, u_ref, o_ref):
    o_ref[...] = jax.lax.dot_general(
        abf_ref[...], u_ref[...].astype(bf16), (((1,), (0,)), ((), ())),
        preferred_element_type=f32)


def _chain3_body(abf_ref, u_ref, r16_ref, h_ref):
    u3 = jax.lax.dot_general(
        abf_ref[...], u_ref[...].astype(bf16), (((1,), (0,)), ((), ())),
        preferred_element_type=f32)
    h_ref[...] = 0.5 * u3 + r16_ref[..., 0:C]


def _final_body(abf_ref, h_ref, r16_ref, o_ref):
    dec = jax.lax.dot_general(
        abf_ref[...], h_ref[...].astype(bf16), (((1,), (0,)), ((), ())),
        preferred_element_type=f32)
    o_ref[...] = 0.5 * dec + r16_ref[..., C:2 * C]


def _full(shape):
    return pl.BlockSpec(shape, lambda i: (0,) * len(shape))


def _rows(b, w):
    return pl.BlockSpec((b, w), lambda i: (i, 0))


def kernel(features, adj, mean_fewshot_logits, W_enc, W_dec, toy_keys,
           toy_labels):
    fw = pl.pallas_call(
        _fw_body,
        grid=(N // BP,),
        in_specs=[_rows(BP, D), _full((D, E))],
        out_specs=_rows(BP, E),
        out_shape=jax.ShapeDtypeStruct((N, E), bf16),
    )(features, W_enc)

    c16 = pl.pallas_call(
        _c16_body,
        grid=(M // BP,),
        in_specs=[_rows(BP, E), _rows(BP, C), _full((C, C)), _full((E, C))],
        out_specs=_rows(BP, 128),
        out_shape=jax.ShapeDtypeStruct((M, 128), f32),
    )(toy_keys, toy_labels, mean_fewshot_logits, W_dec)

    tkb = toy_keys.astype(bf16)

    idx, u0, abf = pl.pallas_call(
        _main_body,
        grid=(N // BA,),
        in_specs=[_rows(BA, N), _full((N, E)), _full((M, E)), _full((E, C))],
        out_specs=[_rows(BA, K), _rows(BA, C), _rows(BA, N)],
        out_shape=[jax.ShapeDtypeStruct((N, K), jnp.int32),
                   jax.ShapeDtypeStruct((N, C), f32),
                   jax.ShapeDtypeStruct((N, N), bf16)],
    )(adj, fw, tkb, W_dec)

    idx_prep = jnp.transpose(idx.reshape(_NW, _ROWS_W, K), (0, 2, 1))
    idx_prep = jnp.pad(idx_prep, ((0, 0), (0, 8 - K), (0, 0)))
    r16 = _sc_gather(idx_prep, c16)

    chain = pl.pallas_call(
        _chain_body,
        grid=(N // BC,),
        in_specs=[_rows(BC, N), _full((N, C))],
        out_specs=_rows(BC, C),
        out_shape=jax.ShapeDtypeStruct((N, C), f32),
    )
    u1 = chain(abf, u0)
    u2 = chain(abf, u1)

    h = pl.pallas_call(
        _chain3_body,
        grid=(N // BC,),
        in_specs=[_rows(BC, N), _full((N, C)), _rows(BC, 16)],
        out_specs=_rows(BC, C),
        out_shape=jax.ShapeDtypeStruct((N, C), f32),
    )(abf, u2, r16)

    out = pl.pallas_call(
        _final_body,
        grid=(N // BC,),
        in_specs=[_rows(BC, N), _full((N, C)), _rows(BC, 16)],
        out_specs=_rows(BC, C),
        out_shape=jax.ShapeDtypeStruct((N, C), f32),
    )(abf, h, r16)
    return out


# speedup vs baseline: 1.0011x; 1.0011x over previous
import functools

import jax
import jax.numpy as jnp
from jax import lax
from jax.experimental import pallas as pl
from jax.experimental.pallas import tpu as pltpu
from jax.experimental.pallas import tpu_sc as plsc

N, D, M, C, E, K = 4096, 256, 8192, 6, 256, 5
BA = 128
BC = 256
BP = 512
NEG = -1e30
BIGI = 2**30
f32 = jnp.float32
bf16 = jnp.bfloat16


def _bmm(a, b, dims):
    return jax.lax.dot_general(
        a.astype(bf16), b.astype(bf16), dims,
        preferred_element_type=f32)


def _fw_body(f_ref, we_ref, fw_ref):
    fw_ref[...] = _bmm(f_ref[...], we_ref[...],
                       (((1,), (0,)), ((), ()))).astype(bf16)


def _c16_body(tk_ref, tl_ref, mfl_ref, wd_ref, c16_ref):
    tv = jax.lax.dot_general(
        tk_ref[...], wd_ref[...], (((1,), (0,)), ((), ())),
        precision=jax.lax.Precision.HIGHEST,
        preferred_element_type=f32) * 0.5
    lab = tl_ref[...]
    m = jnp.max(lab, axis=1, keepdims=True)
    ci = jax.lax.broadcasted_iota(jnp.int32, lab.shape, 1)
    ji = jnp.min(jnp.where(lab == m, ci, BIGI), axis=1, keepdims=True)
    oh = (ci == ji).astype(f32)
    lg = jax.lax.dot_general(
        oh, mfl_ref[...], (((1,), (0,)), ((), ())),
        precision=jax.lax.Precision.HIGHEST,
        preferred_element_type=f32) * 0.1
    z = jnp.zeros((lab.shape[0], 128 - 2 * C), f32)
    c16_ref[...] = jnp.concatenate([tv, lg, z], axis=1)


def _main_body(adj_ref, fw_ref, tk_ref, wd_ref,
               idx_ref, u0_ref, abf_ref):
    a = adj_ref[...]
    rs = jnp.sum(a, axis=1, keepdims=True) + 1e-8
    ab = (a / rs).astype(bf16)
    abf_ref[...] = ab
    p = jnp.maximum(
        jax.lax.dot_general(ab, fw_ref[...], (((1,), (0,)), ((), ())),
                            preferred_element_type=f32), 0.0)
    u0_ref[...] = jax.lax.dot_general(
        p, wd_ref[...], (((1,), (0,)), ((), ())),
        precision=jax.lax.Precision.HIGHEST,
        preferred_element_type=f32)
    s = jax.lax.dot_general(
        p.astype(bf16), tk_ref[...], (((1,), (1,)), ((), ())),
        preferred_element_type=f32)
    col = jax.lax.broadcasted_iota(jnp.int32, s.shape, 1)
    m = jnp.max(s, axis=1, keepdims=True)
    idxs = []
    for k in range(K):
        eq = s == m
        idxs.append(jnp.min(jnp.where(eq, col, BIGI), axis=1, keepdims=True))
        s = jnp.where(eq, NEG, s)
        if k < K - 1:
            m = jnp.max(s, axis=1, keepdims=True)
    idx_ref[...] = jnp.concatenate(idxs, axis=1)


_SC_INFO = plsc.get_sparse_core_info()
_NW = _SC_INFO.num_cores * _SC_INFO.num_subcores
_ROWS_W = N // _NW


def _sc_gather_body(idx_hbm, c16_hbm, out_hbm, idx_v, rows_v, acc_v, sem):
    wid = lax.axis_index("s") * _SC_INFO.num_cores + lax.axis_index("c")
    pltpu.sync_copy(idx_hbm.at[wid], idx_v)
    copies = [pltpu.async_copy(c16_hbm.at[idx_v.at[k]], rows_v.at[k], sem)
              for k in range(K)]
    for cp in copies:
        cp.wait()
    for i in range(_ROWS_W):
        v = rows_v[0, i, pl.ds(0, 16)]
        for k in range(1, K):
            v = v + rows_v[k, i, pl.ds(0, 16)]
        acc_v[i, :] = v
    pltpu.sync_copy(acc_v, out_hbm.at[pl.ds(wid * _ROWS_W, _ROWS_W)])


@functools.partial(
    pl.kernel,
    mesh=plsc.VectorSubcoreMesh(core_axis_name="c", subcore_axis_name="s"),
    out_type=jax.ShapeDtypeStruct((N, 16), f32),
    scratch_types=[
        pltpu.VMEM((8, _ROWS_W), jnp.int32),
        pltpu.VMEM((K, _ROWS_W, 128), f32),
        pltpu.VMEM((_ROWS_W, 16), f32),
        pltpu.SemaphoreType.DMA,
    ],
)
def _sc_gather(idx_hbm, c16_hbm, out_hbm, idx_v, rows_v, acc_v, sem):
    _sc_gather_body(idx_hbm, c16_hbm, out_hbm, idx_v, rows_v, acc_v, sem)


def _chain_body(abf_ref, u_ref, o_ref):
    o_ref[...] = jax.lax.dot_general(
        abf_ref[...], u_ref[...].astype(bf16), (((1,), (0,)), ((), ())),
        preferred_element_type=f32)


def _chain3_body(abf_ref, u_ref, r16_ref, h_ref):
    u3 = jax.lax.dot_general(
        abf_ref[...], u_ref[...].astype(bf16), (((1,), (0,)), ((), ())),
        preferred_element_type=f32)
    h_ref[...] = 0.5 * u3 + r16_ref[..., 0:C]


def _final_body(abf_ref, h_ref, r16_ref, o_ref):
    dec = jax.lax.dot_general(
        abf_ref[...], h_ref[...].astype(bf16), (((1,), (0,)), ((), ())),
        preferred_element_type=f32)
    o_ref[...] = 0.5 * dec + r16_ref[..., C:2 * C]


def _full(shape):
    return pl.BlockSpec(shape, lambda i: (0,) * len(shape))


def _rows(b, w):
    return pl.BlockSpec((b, w), lambda i: (i, 0))


def kernel(features, adj, mean_fewshot_logits, W_enc, W_dec, toy_keys,
           toy_labels):
    fw = pl.pallas_call(
        _fw_body,
        grid=(N // BP,),
        in_specs=[_rows(BP, D), _full((D, E))],
        out_specs=_rows(BP, E),
        out_shape=jax.ShapeDtypeStruct((N, E), bf16),
    )(features, W_enc)

    c16 = pl.pallas_call(
        _c16_body,
        grid=(M // BP,),
        in_specs=[_rows(BP, E), _rows(BP, C), _full((C, C)), _full((E, C))],
        out_specs=_rows(BP, 128),
        out_shape=jax.ShapeDtypeStruct((M, 128), f32),
    )(toy_keys, toy_labels, mean_fewshot_logits, W_dec)

    tkb = toy_keys.astype(bf16)

    idx, u0, abf = pl.pallas_call(
        _main_body,
        grid=(N // BA,),
        in_specs=[_rows(BA, N), _full((N, E)), _full((M, E)), _full((E, C))],
        out_specs=[_rows(BA, K), _rows(BA, C), _rows(BA, N)],
        out_shape=[jax.ShapeDtypeStruct((N, K), jnp.int32),
                   jax.ShapeDtypeStruct((N, C), f32),
                   jax.ShapeDtypeStruct((N, N), bf16)],
    )(adj, fw, tkb, W_dec)

    idx_prep = jnp.transpose(idx.reshape(_NW, _ROWS_W, K), (0, 2, 1))
    idx_prep = jnp.pad(idx_prep, ((0, 0), (0, 8 - K), (0, 0)))
    r16 = _sc_gather(idx_prep, c16)

    chain = pl.pallas_call(
        _chain_body,
        grid=(N // BC,),
        in_specs=[_rows(BC, N), _full((N, C))],
        out_specs=_rows(BC, C),
        out_shape=jax.ShapeDtypeStruct((N, C), f32),
    )
    u1 = chain(abf, u0)
    u2 = chain(abf, u1)

    h = pl.pallas_call(
        _chain3_body,
        grid=(N // BC,),
        in_specs=[_rows(BC, N), _full((N, C)), _rows(BC, 16)],
        out_specs=_rows(BC, C),
        out_shape=jax.ShapeDtypeStruct((N, C), f32),
    )(abf, u2, r16)

    out = pl.pallas_call(
        _final_body,
        grid=(N // BC,),
        in_specs=[_rows(BC, N), _full((N, C)), _rows(BC, 16)],
        out_specs=_rows(BC, C),
        out_shape=jax.ShapeDtypeStruct((N, C), f32),
    )(abf, h, r16)
    return out


# speedup vs baseline: 1.0017x; 1.0006x over previous
import functools

import jax
import jax.numpy as jnp
from jax import lax
from jax.experimental import pallas as pl
from jax.experimental.pallas import tpu as pltpu
from jax.experimental.pallas import tpu_sc as plsc

N, D, M, C, E, K = 4096, 256, 8192, 6, 256, 5
BA = 128
BC = 256
BP = 512
NEG = -1e30
BIGI = 2**30
f32 = jnp.float32
bf16 = jnp.bfloat16


def _bmm(a, b, dims):
    return jax.lax.dot_general(
        a.astype(bf16), b.astype(bf16), dims,
        preferred_element_type=f32)


def _fw_body(f_ref, we_ref, fw_ref):
    fw_ref[...] = _bmm(f_ref[...], we_ref[...],
                       (((1,), (0,)), ((), ()))).astype(bf16)


def _c16_body(tk_ref, tl_ref, mfl_ref, wd_ref, c16_ref):
    tv = jax.lax.dot_general(
        tk_ref[...], wd_ref[...], (((1,), (0,)), ((), ())),
        precision=jax.lax.Precision.HIGHEST,
        preferred_element_type=f32) * 0.5
    lab = tl_ref[...]
    m = jnp.max(lab, axis=1, keepdims=True)
    ci = jax.lax.broadcasted_iota(jnp.int32, lab.shape, 1)
    ji = jnp.min(jnp.where(lab == m, ci, BIGI), axis=1, keepdims=True)
    oh = (ci == ji).astype(f32)
    lg = jax.lax.dot_general(
        oh, mfl_ref[...], (((1,), (0,)), ((), ())),
        precision=jax.lax.Precision.HIGHEST,
        preferred_element_type=f32) * 0.1
    z = jnp.zeros((lab.shape[0], 128 - 2 * C), f32)
    c16_ref[...] = jnp.concatenate([tv, lg, z], axis=1)


def _main_body(adj_ref, fw_ref, tk_ref, wd_ref,
               idx_ref, u0_ref, abf_ref):
    a = adj_ref[...]
    rs = jnp.sum(a, axis=1, keepdims=True) + 1e-8
    ab = (a / rs).astype(bf16)
    abf_ref[...] = ab
    p = jnp.maximum(
        jax.lax.dot_general(ab, fw_ref[...], (((1,), (0,)), ((), ())),
                            preferred_element_type=f32), 0.0)
    u0_ref[...] = jax.lax.dot_general(
        p, wd_ref[...], (((1,), (0,)), ((), ())),
        precision=jax.lax.Precision.HIGHEST,
        preferred_element_type=f32)
    s = jax.lax.dot_general(
        p.astype(bf16), tk_ref[...], (((1,), (1,)), ((), ())),
        preferred_element_type=f32)
    col = jax.lax.broadcasted_iota(jnp.int32, s.shape, 1)
    m = jnp.max(s, axis=1, keepdims=True)
    idxs = []
    for k in range(K):
        eq = s == m
        idxs.append(jnp.min(jnp.where(eq, col, BIGI), axis=1, keepdims=True))
        s = jnp.where(eq, NEG, s)
        if k < K - 1:
            m = jnp.max(s, axis=1, keepdims=True)
    idx_ref[...] = jnp.concatenate(idxs, axis=1)


_SC_INFO = plsc.get_sparse_core_info()
_NW = _SC_INFO.num_cores * _SC_INFO.num_subcores
_ROWS_W = N // _NW


def _sc_gather_body(idx_hbm, c16_hbm, out_hbm, idx_v, rows_v, acc_v, sem):
    wid = lax.axis_index("s") * _SC_INFO.num_cores + lax.axis_index("c")
    pltpu.sync_copy(idx_hbm.at[wid], idx_v)
    copies = [pltpu.async_copy(c16_hbm.at[idx_v.at[k]], rows_v.at[k], sem)
              for k in range(K)]
    for cp in copies:
        cp.wait()
    for i in range(_ROWS_W):
        v = rows_v[0, i, pl.ds(0, 16)]
        for k in range(1, K):
            v = v + rows_v[k, i, pl.ds(0, 16)]
        acc_v[i, :] = v
    pltpu.sync_copy(acc_v, out_hbm.at[pl.ds(wid * _ROWS_W, _ROWS_W)])


@functools.partial(
    pl.kernel,
    mesh=plsc.VectorSubcoreMesh(core_axis_name="c", subcore_axis_name="s"),
    out_type=jax.ShapeDtypeStruct((N, 16), f32),
    scratch_types=[
        pltpu.VMEM((8, _ROWS_W), jnp.int32),
        pltpu.VMEM((K, _ROWS_W, 128), f32),
        pltpu.VMEM((_ROWS_W, 16), f32),
        pltpu.SemaphoreType.DMA,
    ],
)
def _sc_gather(idx_hbm, c16_hbm, out_hbm, idx_v, rows_v, acc_v, sem):
    _sc_gather_body(idx_hbm, c16_hbm, out_hbm, idx_v, rows_v, acc_v, sem)


def _chain_body(abf_ref, u_ref, o_ref):
    o_ref[...] = jax.lax.dot_general(
        abf_ref[...], u_ref[...].astype(bf16), (((1,), (0,)), ((), ())),
        preferred_element_type=f32)


def _chain3_body(abf_ref, u_ref, r16_ref, h_ref):
    u3 = jax.lax.dot_general(
        abf_ref[...], u_ref[...].astype(bf16), (((1,), (0,)), ((), ())),
        preferred_element_type=f32)
    h_ref[...] = 0.5 * u3 + r16_ref[..., 0:C]


def _final_body(abf_ref, h_ref, r16_ref, o_ref):
    dec = jax.lax.dot_general(
        abf_ref[...], h_ref[...].astype(bf16), (((1,), (0,)), ((), ())),
        preferred_element_type=f32)
    o_ref[...] = 0.5 * dec + r16_ref[..., C:2 * C]


def _full(shape):
    return pl.BlockSpec(shape, lambda i: (0,) * len(shape))


def _rows(b, w):
    return pl.BlockSpec((b, w), lambda i: (i, 0))


def kernel(features, adj, mean_fewshot_logits, W_enc, W_dec, toy_keys,
           toy_labels):
    fw = pl.pallas_call(
        _fw_body,
        grid=(N // BP,),
        in_specs=[_rows(BP, D), _full((D, E))],
        out_specs=_rows(BP, E),
        out_shape=jax.ShapeDtypeStruct((N, E), bf16),
    )(features, W_enc)

    c16 = pl.pallas_call(
        _c16_body,
        grid=(M // BP,),
        in_specs=[_rows(BP, E), _rows(BP, C), _full((C, C)), _full((E, C))],
        out_specs=_rows(BP, 128),
        out_shape=jax.ShapeDtypeStruct((M, 128), f32),
    )(toy_keys, toy_labels, mean_fewshot_logits, W_dec)

    tkb = toy_keys.astype(bf16)

    idx, u0, abf = pl.pallas_call(
        _main_body,
        grid=(N // BA,),
        in_specs=[_rows(BA, N), _full((N, E)), _full((M, E)), _full((E, C))],
        out_specs=[_rows(BA, K), _rows(BA, C), _rows(BA, N)],
        out_shape=[jax.ShapeDtypeStruct((N, K), jnp.int32),
                   jax.ShapeDtypeStruct((N, C), f32),
                   jax.ShapeDtypeStruct((N, N), bf16)],
    )(adj, fw, tkb, W_dec)

    chain = pl.pallas_call(
        _chain_body,
        grid=(N // BC,),
        in_specs=[_rows(BC, N), _full((N, C))],
        out_specs=_rows(BC, C),
        out_shape=jax.ShapeDtypeStruct((N, C), f32),
    )
    u1 = chain(abf, u0)

    idx_prep = jnp.transpose(idx.reshape(_NW, _ROWS_W, K), (0, 2, 1))
    idx_prep = jnp.pad(idx_prep, ((0, 0), (0, 8 - K), (0, 0)))
    r16 = _sc_gather(idx_prep, c16)

    u2 = chain(abf, u1)

    h = pl.pallas_call(
        _chain3_body,
        grid=(N // BC,),
        in_specs=[_rows(BC, N), _full((N, C)), _rows(BC, 16)],
        out_specs=_rows(BC, C),
        out_shape=jax.ShapeDtypeStruct((N, C), f32),
    )(abf, u2, r16)

    out = pl.pallas_call(
        _final_body,
        grid=(N // BC,),
        in_specs=[_rows(BC, N), _full((N, C)), _rows(BC, 16)],
        out_specs=_rows(BC, C),
        out_shape=jax.ShapeDtypeStruct((N, C), f32),
    )(abf, h, r16)
    return out
